# Initial kernel scaffold; baseline (speedup 1.0000x reference)
import jax
import jax.numpy as jnp
from jax.experimental import pallas as pl


def kernel(x, word_table, temp_table):
    raise NotImplementedError("write your pallas kernel here")


# speedup vs baseline: 1.0369x; 1.0369x over previous
import jax
import jax.numpy as jnp
from jax.experimental import pallas as pl
from jax.experimental.pallas import tpu as pltpu


def _body(x_ref, w_ref, t_ref, o_ref):
    o_ref[...] = x_ref[...] + w_ref[...][None, :, None, :] + t_ref[...][None, None, :, :]


def kernel(x, word_table, temp_table):
    B, W, T, H = x.shape
    grid = (B,)
    return pl.pallas_call(
        _body,
        grid=grid,
        in_specs=[
            pl.BlockSpec((1, W, T, H), lambda b: (b, 0, 0, 0)),
            pl.BlockSpec((W, H), lambda b: (0, 0)),
            pl.BlockSpec((T, H), lambda b: (0, 0)),
        ],
        out_specs=pl.BlockSpec((1, W, T, H), lambda b: (b, 0, 0, 0)),
        out_shape=jax.ShapeDtypeStruct((B, W, T, H), x.dtype),
    )(x, word_table, temp_table)
